# baseline (speedup 1.0000x reference)
import functools

import jax
import jax.numpy as jnp
from jax import lax
from jax.experimental import pallas as pl
from jax.experimental.pallas import tpu as pltpu
from jax.experimental.pallas import tpu_sc as plsc

VOCAB = 100000
MAX_POS = 8192
D_MODEL = 768
BATCH = 4
SEQ = 2048

_info = plsc.get_sparse_core_info()
NC, NS, NL = _info.num_cores, _info.num_subcores, _info.num_lanes
NW = NC * NS
S_PER_W = SEQ // NW
VPR = D_MODEL // NL
CH = 32
NT = BATCH * (S_PER_W // CH)
NBUF = 4
_PK = None


def _emb_body(ids_hbm, tok_hbm, pos_hbm, out_hbm, idx_all, pos_bf, tok_bufs,
              isem, gsem0, gsem1, gsem2, gsem3, wsem0, wsem1, wsem2, wsem3):
    gsems = (gsem0, gsem1, gsem2, gsem3)
    wsems = (wsem0, wsem1, wsem2, wsem3)
    wid = lax.axis_index("s") * NC + lax.axis_index("c")
    s0 = wid * S_PER_W
    himask = jnp.uint32(0xFFFF0000)
    half = jnp.uint32(0x8000)

    idx_d = [
        pltpu.async_copy(ids_hbm.at[b, pl.ds(s0, S_PER_W)],
                         idx_all.at[b], isem)
        for b in range(BATCH)
    ]

    gd = [None] * NT
    wd = [None] * NT

    def start_gather(t):
        b, c = divmod(t, S_PER_W // CH)
        gd[t] = pltpu.async_copy(
            tok_hbm.at[idx_all.at[b, pl.ds(c * CH, CH)]],
            tok_bufs.at[t % NBUF], gsems[t % NBUF])

    for d in idx_d:
        d.wait()
    start_gather(0)
    start_gather(1)
    start_gather(2)

    for h in range(S_PER_W // CH):
        pltpu.sync_copy(pos_hbm.at[pl.ds(s0 + h * CH, CH)], tok_bufs.at[3])

        def prow(r, _, _h=h):
            for k in range(VPR // 2):
                a = lax.bitcast_convert_type(
                    tok_bufs[3, r, pl.ds(2 * k * NL, NL)], jnp.uint32)
                b = lax.bitcast_convert_type(
                    tok_bufs[3, r, pl.ds((2 * k + 1) * NL, NL)], jnp.uint32)
                lo = lax.shift_right_logical(a + half, jnp.uint32(16))
                hi = (b + half) & himask
                pos_bf[_h * CH + r, pl.ds(k * NL, NL)] = lo | hi
            return 0

        lax.fori_loop(0, CH, prow, 0)

    start_gather(3)

    waited = set()
    for t in range(NT):
        b, c = divmod(t, S_PER_W // CH)
        gd[t].wait()

        def row(r, _, _t=t, _c=c):
            for k in range(VPR // 2):
                y = pos_bf[_c * CH + r, pl.ds(k * NL, NL)]
                a = lax.bitcast_convert_type(
                    lax.shift_left(y, jnp.uint32(16)), jnp.float32)
                b2 = lax.bitcast_convert_type(y & himask, jnp.float32)
                plsc.addupdate(
                    tok_bufs.at[_t % NBUF, r, pl.ds(2 * k * NL, NL)], a)
                plsc.addupdate(
                    tok_bufs.at[_t % NBUF, r, pl.ds((2 * k + 1) * NL, NL)], b2)
            return 0

        lax.fori_loop(0, CH, row, 0)

        if t + 3 < NT:
            if t - 1 >= 0:
                wd[t - 1].wait()
                waited.add(t - 1)
            start_gather(t + 3)

        wd[t] = pltpu.async_copy(
            tok_bufs.at[t % NBUF],
            out_hbm.at[b, pl.ds(s0 + c * CH, CH)], wsems[t % NBUF])

    for t in range(NT):
        if t not in waited:
            wd[t].wait()


_emb = functools.partial(
    pl.kernel,
    out_type=jax.ShapeDtypeStruct((BATCH, SEQ, D_MODEL), jnp.float32),
    mesh=plsc.VectorSubcoreMesh(core_axis_name="c", subcore_axis_name="s"),
    scratch_types=[
        pltpu.VMEM((BATCH, S_PER_W), jnp.int32),
        pltpu.VMEM((S_PER_W, D_MODEL // 2), jnp.uint32),
        pltpu.VMEM((NBUF, CH, D_MODEL), jnp.float32),
    ] + [pltpu.SemaphoreType.DMA] * 9,
)(_emb_body)


@jax.jit
def kernel(input_ids, token_table, pos_table):
    return _emb(input_ids.astype(jnp.int32), token_table, pos_table)


# speedup vs baseline: 1.2103x; 1.2103x over previous
import functools

import jax
import jax.numpy as jnp
from jax import lax
from jax.experimental import pallas as pl
from jax.experimental.pallas import tpu as pltpu
from jax.experimental.pallas import tpu_sc as plsc

VOCAB = 100000
MAX_POS = 8192
D_MODEL = 768
BATCH = 4
SEQ = 2048

_info = plsc.get_sparse_core_info()
NC, NS, NL = _info.num_cores, _info.num_subcores, _info.num_lanes
NW = NC * NS
S_PER_W = SEQ // NW
VPR = D_MODEL // NL
CH = 32
NT = BATCH * (S_PER_W // CH)
NBUF = 3


def _emb_body(ids_hbm, tok_hbm, pos_hbm, out_hbm, idx_all, pos_buf, tok_bufs,
              isem, gsem0, gsem1, gsem2, wsem0, wsem1, wsem2):
    gsems = (gsem0, gsem1, gsem2)
    wsems = (wsem0, wsem1, wsem2)
    wid = lax.axis_index("s") * NC + lax.axis_index("c")
    s0 = wid * S_PER_W

    idx_d = [
        pltpu.async_copy(ids_hbm.at[b, pl.ds(s0, S_PER_W)],
                         idx_all.at[b], isem)
        for b in range(BATCH)
    ]
    pos_d = pltpu.async_copy(pos_hbm.at[pl.ds(s0, S_PER_W)], pos_buf, isem)

    gd = [None] * NT
    wd = [None] * NT

    def start_gather(t):
        b, c = divmod(t, S_PER_W // CH)
        gd[t] = pltpu.async_copy(
            tok_hbm.at[idx_all.at[b, pl.ds(c * CH, CH)]],
            tok_bufs.at[t % NBUF], gsems[t % NBUF])

    for d in idx_d:
        d.wait()
    for t in range(NBUF):
        start_gather(t)
    pos_d.wait()

    waited = set()
    for t in range(NT):
        b, c = divmod(t, S_PER_W // CH)
        gd[t].wait()

        @plsc.parallel_loop(0, CH, 1, unroll=4)
        def row(r, _t=t, _c=c):
            for k in range(VPR):
                x = pos_buf[_c * CH + r, pl.ds(k * NL, NL)]
                plsc.addupdate(
                    tok_bufs.at[_t % NBUF, r, pl.ds(k * NL, NL)], x)

        if t >= 1 and t + NBUF - 1 < NT:
            wd[t - 1].wait()
            waited.add(t - 1)
            start_gather(t + NBUF - 1)

        wd[t] = pltpu.async_copy(
            tok_bufs.at[t % NBUF],
            out_hbm.at[b, pl.ds(s0 + c * CH, CH)], wsems[t % NBUF])

    for t in range(NT):
        if t not in waited:
            wd[t].wait()


_emb = functools.partial(
    pl.kernel,
    out_type=jax.ShapeDtypeStruct((BATCH, SEQ, D_MODEL), jnp.float32),
    mesh=plsc.VectorSubcoreMesh(core_axis_name="c", subcore_axis_name="s"),
    scratch_types=[
        pltpu.VMEM((BATCH, S_PER_W), jnp.int32),
        pltpu.VMEM((S_PER_W, D_MODEL), jnp.float32),
        pltpu.VMEM((NBUF, CH, D_MODEL), jnp.float32),
    ] + [pltpu.SemaphoreType.DMA] * 7,
)(_emb_body)


@jax.jit
def kernel(input_ids, token_table, pos_table):
    return _emb(input_ids.astype(jnp.int32), token_table, pos_table)


# speedup vs baseline: 1.2339x; 1.0195x over previous
import functools

import jax
import jax.numpy as jnp
from jax import lax
from jax.experimental import pallas as pl
from jax.experimental.pallas import tpu as pltpu
from jax.experimental.pallas import tpu_sc as plsc

VOCAB = 100000
MAX_POS = 8192
D_MODEL = 768
BATCH = 4
SEQ = 2048

_info = plsc.get_sparse_core_info()
NC, NS, NL = _info.num_cores, _info.num_subcores, _info.num_lanes
NW = NC * NS
S_PER_W = SEQ // NW
VPR = D_MODEL // NL
CH = 32
NT = BATCH * (S_PER_W // CH)
NBUF = 3


def _emb_body(ids_hbm, tok_hbm, pos_hbm, out_hbm, idx_all, pos_buf, tok_bufs,
              isem, gsem0, gsem1, gsem2, wsem0, wsem1, wsem2):
    gsems = (gsem0, gsem1, gsem2)
    wsems = (wsem0, wsem1, wsem2)
    wid = lax.axis_index("s") * NC + lax.axis_index("c")
    s0 = wid * S_PER_W

    idx_d = [
        pltpu.async_copy(ids_hbm.at[b, pl.ds(s0, S_PER_W)],
                         idx_all.at[b], isem)
        for b in range(BATCH)
    ]
    pos_d = pltpu.async_copy(pos_hbm.at[pl.ds(s0, S_PER_W)], pos_buf, isem)

    gd = [None] * NT
    wd = [None] * NT

    def start_gather(t):
        b, c = divmod(t, S_PER_W // CH)
        gd[t] = pltpu.async_copy(
            tok_hbm.at[idx_all.at[b, pl.ds(c * CH, CH)]],
            tok_bufs.at[t % NBUF], gsems[t % NBUF])

    for d in idx_d:
        d.wait()
    start_gather(0)
    start_gather(1)
    pos_d.wait()

    waited = set()
    for t in range(NT):
        b, c = divmod(t, S_PER_W // CH)
        gd[t].wait()

        @plsc.parallel_loop(0, CH, 1, unroll=4)
        def row(r, _t=t, _c=c):
            for k in range(VPR):
                x = pos_buf[_c * CH + r, pl.ds(k * NL, NL)]
                plsc.addupdate(
                    tok_bufs.at[_t % NBUF, r, pl.ds(k * NL, NL)], x)

        if t + 2 < NT:
            if t - 1 >= 0:
                wd[t - 1].wait()
                waited.add(t - 1)
            start_gather(t + 2)

        wd[t] = pltpu.async_copy(
            tok_bufs.at[t % NBUF],
            out_hbm.at[b, pl.ds(s0 + c * CH, CH)], wsems[t % NBUF])

    for t in range(NT):
        if t not in waited:
            wd[t].wait()


_emb = functools.partial(
    pl.kernel,
    out_type=jax.ShapeDtypeStruct((BATCH, SEQ, D_MODEL), jnp.float32),
    mesh=plsc.VectorSubcoreMesh(core_axis_name="c", subcore_axis_name="s"),
    scratch_types=[
        pltpu.VMEM((BATCH, S_PER_W), jnp.int32),
        pltpu.VMEM((S_PER_W, D_MODEL), jnp.float32),
        pltpu.VMEM((NBUF, CH, D_MODEL), jnp.float32),
    ] + [pltpu.SemaphoreType.DMA] * 7,
)(_emb_body)


@jax.jit
def kernel(input_ids, token_table, pos_table):
    return _emb(input_ids.astype(jnp.int32), token_table, pos_table)


# speedup vs baseline: 1.3042x; 1.0570x over previous
import functools

import jax
import jax.numpy as jnp
from jax import lax
from jax.experimental import pallas as pl
from jax.experimental.pallas import tpu as pltpu
from jax.experimental.pallas import tpu_sc as plsc

VOCAB = 100000
MAX_POS = 8192
D_MODEL = 768
BATCH = 4
SEQ = 2048

_info = plsc.get_sparse_core_info()
NC, NS, NL = _info.num_cores, _info.num_subcores, _info.num_lanes
NW = NC * NS
S_PER_W = SEQ // NW
VPR = D_MODEL // NL
CH = 32
NT = BATCH * (S_PER_W // CH)
NBUF = 3


def _emb_body(ids_hbm, tok_hbm, pos_hbm, out_hbm, idx_all, pos_buf, tok_bufs,
              isem, gsem0, gsem1, gsem2, wsem0, wsem1, wsem2):
    gsems = (gsem0, gsem1, gsem2)
    wsems = (wsem0, wsem1, wsem2)
    wid = lax.axis_index("s") * NC + lax.axis_index("c")
    s0 = wid * S_PER_W

    idx_d = [
        pltpu.async_copy(ids_hbm.at[b, pl.ds(s0, S_PER_W)],
                         idx_all.at[b], isem)
        for b in range(BATCH)
    ]
    pos_d = pltpu.async_copy(pos_hbm.at[pl.ds(s0, S_PER_W)], pos_buf, isem)

    gd = [None] * NT
    wd = [None] * NT

    def start_gather(t):
        b, c = divmod(t, S_PER_W // CH)
        gd[t] = pltpu.async_copy(
            tok_hbm.at[idx_all.at[b, pl.ds(c * CH, CH)]],
            tok_bufs.at[t % NBUF], gsems[t % NBUF])

    for d in idx_d:
        d.wait()
    start_gather(0)
    start_gather(1)
    pos_d.wait()

    waited = set()
    for t in range(NT):
        b, c = divmod(t, S_PER_W // CH)
        gd[t].wait()

        @plsc.parallel_loop(0, CH, 1, unroll=8)
        def row(r, _t=t, _c=c):
            for k in range(VPR):
                x = pos_buf[_c * CH + r, pl.ds(k * NL, NL)]
                plsc.addupdate(
                    tok_bufs.at[_t % NBUF, r, pl.ds(k * NL, NL)], x)

        if t + 2 < NT:
            if t - 1 >= 0:
                wd[t - 1].wait()
                waited.add(t - 1)
            start_gather(t + 2)

        wd[t] = pltpu.async_copy(
            tok_bufs.at[t % NBUF],
            out_hbm.at[b, pl.ds(s0 + c * CH, CH)], wsems[t % NBUF])

    for t in range(NT):
        if t not in waited:
            wd[t].wait()


_emb = functools.partial(
    pl.kernel,
    out_type=jax.ShapeDtypeStruct((BATCH, SEQ, D_MODEL), jnp.float32),
    mesh=plsc.VectorSubcoreMesh(core_axis_name="c", subcore_axis_name="s"),
    scratch_types=[
        pltpu.VMEM((BATCH, S_PER_W), jnp.int32),
        pltpu.VMEM((S_PER_W, D_MODEL), jnp.float32),
        pltpu.VMEM((NBUF, CH, D_MODEL), jnp.float32),
    ] + [pltpu.SemaphoreType.DMA] * 7,
)(_emb_body)


@jax.jit
def kernel(input_ids, token_table, pos_table):
    return _emb(input_ids.astype(jnp.int32), token_table, pos_table)


# speedup vs baseline: 1.3069x; 1.0021x over previous
import functools

import jax
import jax.numpy as jnp
from jax import lax
from jax.experimental import pallas as pl
from jax.experimental.pallas import tpu as pltpu
from jax.experimental.pallas import tpu_sc as plsc

VOCAB = 100000
MAX_POS = 8192
D_MODEL = 768
BATCH = 4
SEQ = 2048

_info = plsc.get_sparse_core_info()
NC, NS, NL = _info.num_cores, _info.num_subcores, _info.num_lanes
NW = NC * NS
S_PER_W = SEQ // NW
VPR = D_MODEL // NL
CH = 32
NT = BATCH * (S_PER_W // CH)
NBUF = 3


def _emb_body(ids_hbm, tok_hbm, pos_hbm, out_hbm, idx_all, pos_buf, tok_bufs,
              isem, gsem0, gsem1, gsem2, wsem0, wsem1, wsem2):
    gsems = (gsem0, gsem1, gsem2)
    wsems = (wsem0, wsem1, wsem2)
    wid = lax.axis_index("s") * NC + lax.axis_index("c")
    s0 = wid * S_PER_W

    idx_d = [
        pltpu.async_copy(ids_hbm.at[b, pl.ds(s0, S_PER_W)],
                         idx_all.at[b], isem)
        for b in range(BATCH)
    ]
    pos_d = pltpu.async_copy(pos_hbm.at[pl.ds(s0, S_PER_W)], pos_buf, isem)

    gd = [None] * NT
    wd = [None] * NT

    def start_gather(t):
        b, c = divmod(t, S_PER_W // CH)
        gd[t] = pltpu.async_copy(
            tok_hbm.at[idx_all.at[b, pl.ds(c * CH, CH)]],
            tok_bufs.at[t % NBUF], gsems[t % NBUF])

    for d in idx_d:
        d.wait()
    start_gather(0)
    start_gather(1)
    pos_d.wait()

    waited = set()
    for t in range(NT):
        b, c = divmod(t, S_PER_W // CH)
        gd[t].wait()

        @plsc.parallel_loop(0, CH, 1, unroll=8)
        def row(r, _t=t, _c=c):
            for k in range(VPR):
                x = pos_buf[_c * CH + r, pl.ds(k * NL, NL)]
                plsc.addupdate(
                    tok_bufs.at[_t % NBUF, r, pl.ds(k * NL, NL)], x)

        wd[t] = pltpu.async_copy(
            tok_bufs.at[t % NBUF],
            out_hbm.at[b, pl.ds(s0 + c * CH, CH)], wsems[t % NBUF])

        if t + 2 < NT:
            if t - 1 >= 0:
                wd[t - 1].wait()
                waited.add(t - 1)
            start_gather(t + 2)

    for t in range(NT):
        if t not in waited:
            wd[t].wait()


_emb = functools.partial(
    pl.kernel,
    out_type=jax.ShapeDtypeStruct((BATCH, SEQ, D_MODEL), jnp.float32),
    mesh=plsc.VectorSubcoreMesh(core_axis_name="c", subcore_axis_name="s"),
    scratch_types=[
        pltpu.VMEM((BATCH, S_PER_W), jnp.int32),
        pltpu.VMEM((S_PER_W, D_MODEL), jnp.float32),
        pltpu.VMEM((NBUF, CH, D_MODEL), jnp.float32),
    ] + [pltpu.SemaphoreType.DMA] * 7,
)(_emb_body)


@jax.jit
def kernel(input_ids, token_table, pos_table):
    return _emb(input_ids.astype(jnp.int32), token_table, pos_table)
